# Initial kernel scaffold; baseline (speedup 1.0000x reference)
import jax
import jax.numpy as jnp
from jax.experimental import pallas as pl


def kernel(embeddings, nodes, clusters, W1, b1, W2, b2, V1, c1, V2, c2):
    raise NotImplementedError("write your pallas kernel here")


# speedup vs baseline: 11.6277x; 11.6277x over previous
import functools

import jax
import jax.numpy as jnp
from jax import lax
from jax.experimental import pallas as pl

N = 50000
C = 1024
EMB = 16
LATENT = 128
HIDDEN = 128
KS = 8
KB = 4
B = 1024
NB = 49
NPAD = B * NB
CB = 128
EB = CB * KS

_f32 = jnp.float32
NEG = float("-inf")


def _dot(a, b, dims):
    return lax.dot_general(a, b, (dims, ((), ())),
                           preferred_element_type=_f32,
                           precision=lax.Precision.HIGHEST)


def _dot16(a, b, dims):
    return lax.dot_general(a.astype(jnp.bfloat16), b.astype(jnp.bfloat16),
                           (dims, ((), ())), preferred_element_type=_f32)


def _ln(x, eps=1e-5):
    mu = jnp.mean(x, axis=-1, keepdims=True)
    var = jnp.mean((x - mu) ** 2, axis=-1, keepdims=True)
    return (x - mu) / jnp.sqrt(var + eps)


def _k1(clus_ref, emb_ref, sums_ref, counts_ref, embn_ref):
    i = pl.program_id(0)
    x = emb_ref[...]
    nrm = jnp.sqrt(jnp.sum(x * x, axis=1, keepdims=True))
    e = x / (nrm + 1e-12)
    embn_ref[...] = e
    clus = clus_ref[...]
    iota_c = lax.broadcasted_iota(jnp.int32, (B, C), 1)
    valid = (i * B + lax.broadcasted_iota(jnp.int32, (B, 1), 0)) < N
    oh = jnp.where((clus == iota_c) & valid, 1.0, 0.0)

    @pl.when(i == 0)
    def _():
        sums_ref[...] = jnp.zeros_like(sums_ref)
        counts_ref[...] = jnp.zeros_like(counts_ref)

    sums_ref[...] += _dot(oh, e, ((0,), (0,)))
    counts_ref[...] += _dot(oh, jnp.ones((B, 1), _f32), ((0,), (0,)))


def _kmeans(sums_ref, counts_ref, means_ref):
    s = sums_ref[...]
    c = counts_ref[...]
    m = s / jnp.maximum(c, 1.0)
    nrm = jnp.sqrt(jnp.sum(m * m, axis=1, keepdims=True))
    means_ref[...] = m / (nrm + 1e-12)


def _k2(embn_ref, means_ref, bidx_ref, bw_ref, bden_ref):
    i = pl.program_id(0)
    e = embn_ref[...]
    m = means_ref[...]
    sim = _dot16(e, m, ((1,), (1,)))
    iota_c = lax.broadcasted_iota(jnp.int32, (B, C), 1)
    valid = (i * B + lax.broadcasted_iota(jnp.int32, (B, 1), 0)) < N
    x = sim
    idxs, ws = [], []
    for _ in range(KB):
        mx = jnp.max(x, axis=1, keepdims=True)
        idx = jnp.min(jnp.where(x == mx, iota_c, C), axis=1, keepdims=True)
        x = jnp.where(iota_c == idx, NEG, x)
        idxs.append(idx)
        ws.append(jnp.exp(mx))
    bidx = jnp.concatenate(idxs, axis=1)
    bw = jnp.where(valid, jnp.concatenate(ws, axis=1), 0.0)
    bidx_ref[...] = bidx
    bw_ref[...] = bw
    acc = jnp.zeros((1, C), _f32)
    for k in range(KB):
        acc += jnp.sum(jnp.where(iota_c == bidx[:, k:k + 1], bw[:, k:k + 1], 0.0),
                       axis=0, keepdims=True)

    @pl.when(i == 0)
    def _():
        bden_ref[...] = jnp.zeros_like(bden_ref)

    bden_ref[...] += acc


def _k3(nodes_ref, bidx_ref, bw_ref, bden_ref, snin_ref, bew_ref):
    i = pl.program_id(0)
    nd = nodes_ref[...]
    l1 = jnp.sum(jnp.abs(nd), axis=1, keepdims=True)
    nl = nd / (l1 + 1e-12)
    bidx = bidx_ref[...]
    bw = bw_ref[...]
    den = bden_ref[...]
    iota_c = lax.broadcasted_iota(jnp.int32, (B, C), 1)
    S = jnp.zeros((B, C), _f32)
    wn_cols = []
    for k in range(KB):
        ohb = iota_c == bidx[:, k:k + 1]
        dg = jnp.sum(jnp.where(ohb, den, 0.0), axis=1, keepdims=True)
        wn = bw[:, k:k + 1] / (dg + 1e-12)
        wn_cols.append(wn)
        S = S + jnp.where(ohb, wn, 0.0)
    bew_ref[...] = jnp.concatenate(wn_cols, axis=1)

    @pl.when(i == 0)
    def _():
        snin_ref[...] = jnp.zeros_like(snin_ref)

    snin_ref[...] += _dot(S, nl, ((0,), (0,)))


def _k4a(means_ref, snin_ref, W1_ref, b1_ref, W2_ref, b2_ref,
         sn_ref, sidx_ref, sew1_ref, sew2_ref):
    m = means_ref[...]
    sim = _dot16(m, m, ((1,), (1,)))
    iota_r = lax.broadcasted_iota(jnp.int32, (C, C), 0)
    iota_c = lax.broadcasted_iota(jnp.int32, (C, C), 1)
    x = sim
    idxs, vals = [], []
    for _ in range(KS):
        mx = jnp.max(x, axis=1, keepdims=True)
        idx = jnp.min(jnp.where(x == mx, iota_c, C), axis=1, keepdims=True)
        x = jnp.where(iota_c == idx, NEG, x)
        idxs.append(idx)
        vals.append(mx)
    sidx = jnp.concatenate(idxs, axis=1)
    svals = jnp.concatenate(vals, axis=1)
    sw = jax.nn.sigmoid(svals)
    sidx_ref[...] = sidx
    rowsum = jnp.sum(sw, axis=1, keepdims=True)
    A = jnp.where(iota_r == iota_c, rowsum, 0.0)
    for k in range(KS):
        A = A + jnp.where(iota_c == sidx[:, k:k + 1], sw[:, k:k + 1], 0.0)
    ones_c1 = jnp.ones((C, 1), _f32)
    sden = _dot(A, ones_c1, ((0,), (0,)))
    sew1_ref[...] = sw / (sden + 1e-12)
    gcols = []
    for k in range(KS):
        ohf = jnp.where(iota_c == sidx[:, k:k + 1], 1.0, 0.0)
        gcols.append(_dot(ohf, sden, ((1,), (0,))))
    G = jnp.concatenate(gcols, axis=1)
    sew2_ref[...] = sw / (G + 1e-12)
    snin = snin_ref[...]
    h = jax.nn.relu(_ln(_dot(snin, W1_ref[...], ((1,), (0,))) + b1_ref[...]))
    p = jax.nn.relu(_ln(_dot16(h, W2_ref[...], ((1,), (0,))) + b2_ref[...]))
    sn_ref[...] = jnp.concatenate([m, p], axis=1)


def _k4b(sn_ref, snb_ref, sidx_ref, V1a_ref, V1b_ref, c1_ref, V2_ref, c2_ref,
         out1_ref, out2_ref):
    sn = sn_ref[...]
    snb = snb_ref[...]
    sidx_f = sidx_ref[...].astype(_f32)
    iota_er = lax.broadcasted_iota(jnp.int32, (EB, CB), 0)
    iota_ec = lax.broadcasted_iota(jnp.int32, (EB, CB), 1)
    ohrep = jnp.where(iota_er // KS == iota_ec, 1.0, 0.0)
    rep = _dot(ohrep, snb, ((1,), (0,)))
    kmod = iota_er[:, :1] % KS
    sidx_exp = jnp.zeros((EB, 1), _f32)
    for k in range(KS):
        col = _dot(ohrep, sidx_f[:, k:k + 1], ((1,), (0,)))
        sidx_exp = jnp.where(kmod == k, col, sidx_exp)
    sidx_e = sidx_exp.astype(jnp.int32)
    iota_gc = lax.broadcasted_iota(jnp.int32, (EB, C), 1)
    ohg = jnp.where(iota_gc == sidx_e, 1.0, 0.0)
    gat = _dot(ohg, sn, ((1,), (0,)))
    V1a = V1a_ref[...]
    V1b = V1b_ref[...]
    G1 = _dot16(gat, V1a, ((1,), (0,)))
    G2 = _dot16(gat, V1b, ((1,), (0,)))
    R1 = _dot16(rep, V1a, ((1,), (0,)))
    R2 = _dot16(rep, V1b, ((1,), (0,)))
    c1 = c1_ref[...]
    c2 = c2_ref[...]
    V2 = V2_ref[...]
    h1 = jax.nn.relu(_ln(G1 + R2 + c1))
    out1_ref[...] = jax.nn.relu(_ln(_dot16(h1, V2, ((1,), (0,))) + c2))
    h2 = jax.nn.relu(_ln(R1 + G2 + c1))
    out2_ref[...] = jax.nn.relu(_ln(_dot16(h2, V2, ((1,), (0,))) + c2))


def kernel(embeddings, nodes, clusters, W1, b1, W2, b2, V1, c1, V2, c2):
    pad = NPAD - N
    embp = jnp.pad(embeddings, ((0, pad), (0, 0)))
    nodesp = jnp.pad(nodes, ((0, pad), (0, 0)))
    clusp = jnp.pad(clusters.astype(jnp.int32), (0, pad)).reshape(NPAD, 1)

    sums, counts, embn = pl.pallas_call(
        _k1,
        grid=(NB,),
        in_specs=[pl.BlockSpec((B, 1), lambda i: (i, 0)),
                  pl.BlockSpec((B, EMB), lambda i: (i, 0))],
        out_specs=[pl.BlockSpec((C, EMB), lambda i: (0, 0)),
                   pl.BlockSpec((C, 1), lambda i: (0, 0)),
                   pl.BlockSpec((B, EMB), lambda i: (i, 0))],
        out_shape=[jax.ShapeDtypeStruct((C, EMB), _f32),
                   jax.ShapeDtypeStruct((C, 1), _f32),
                   jax.ShapeDtypeStruct((NPAD, EMB), _f32)],
    )(clusp, embp)

    means = pl.pallas_call(
        _kmeans,
        out_shape=jax.ShapeDtypeStruct((C, EMB), _f32),
    )(sums, counts)

    bidx, bw, bden = pl.pallas_call(
        _k2,
        grid=(NB,),
        in_specs=[pl.BlockSpec((B, EMB), lambda i: (i, 0)),
                  pl.BlockSpec((C, EMB), lambda i: (0, 0))],
        out_specs=[pl.BlockSpec((B, KB), lambda i: (i, 0)),
                   pl.BlockSpec((B, KB), lambda i: (i, 0)),
                   pl.BlockSpec((1, C), lambda i: (0, 0))],
        out_shape=[jax.ShapeDtypeStruct((NPAD, KB), jnp.int32),
                   jax.ShapeDtypeStruct((NPAD, KB), _f32),
                   jax.ShapeDtypeStruct((1, C), _f32)],
    )(embn, means)

    snin, bew = pl.pallas_call(
        _k3,
        grid=(NB,),
        in_specs=[pl.BlockSpec((B, LATENT), lambda i: (i, 0)),
                  pl.BlockSpec((B, KB), lambda i: (i, 0)),
                  pl.BlockSpec((B, KB), lambda i: (i, 0)),
                  pl.BlockSpec((1, C), lambda i: (0, 0))],
        out_specs=[pl.BlockSpec((C, LATENT), lambda i: (0, 0)),
                   pl.BlockSpec((B, KB), lambda i: (i, 0))],
        out_shape=[jax.ShapeDtypeStruct((C, LATENT), _f32),
                   jax.ShapeDtypeStruct((NPAD, KB), _f32)],
    )(nodesp, bidx, bw, bden)

    sn, sidx, sew1, sew2 = pl.pallas_call(
        _k4a,
        out_shape=[jax.ShapeDtypeStruct((C, LATENT), _f32),
                   jax.ShapeDtypeStruct((C, KS), jnp.int32),
                   jax.ShapeDtypeStruct((C, KS), _f32),
                   jax.ShapeDtypeStruct((C, KS), _f32)],
    )(means, snin, W1, b1.reshape(1, HIDDEN), W2, b2.reshape(1, LATENT - EMB))

    NB4 = C // CB
    out1, out2 = pl.pallas_call(
        _k4b,
        grid=(NB4,),
        in_specs=[pl.BlockSpec((C, LATENT), lambda i: (0, 0)),
                  pl.BlockSpec((CB, LATENT), lambda i: (i, 0)),
                  pl.BlockSpec((CB, KS), lambda i: (i, 0)),
                  pl.BlockSpec((LATENT, HIDDEN), lambda i: (0, 0)),
                  pl.BlockSpec((LATENT, HIDDEN), lambda i: (0, 0)),
                  pl.BlockSpec((1, HIDDEN), lambda i: (0, 0)),
                  pl.BlockSpec((HIDDEN, LATENT), lambda i: (0, 0)),
                  pl.BlockSpec((1, LATENT), lambda i: (0, 0))],
        out_specs=[pl.BlockSpec((EB, LATENT), lambda i: (i, 0)),
                   pl.BlockSpec((EB, LATENT), lambda i: (i, 0))],
        out_shape=[jax.ShapeDtypeStruct((C * KS, LATENT), _f32),
                   jax.ShapeDtypeStruct((C * KS, LATENT), _f32)],
    )(sn, sn, sidx, V1[:LATENT], V1[LATENT:], c1.reshape(1, HIDDEN),
      V2, c2.reshape(1, LATENT))

    supernodes = sn
    superedges = jnp.concatenate([out1, out2], axis=0)
    bipartite_edge_weights = bew[:N].reshape(-1)
    super_edge_weights = jnp.concatenate([sew1.reshape(-1), sew2.reshape(-1)])
    return supernodes, superedges, bipartite_edge_weights, super_edge_weights
